# Initial kernel scaffold; baseline (speedup 1.0000x reference)
import jax
import jax.numpy as jnp
from jax.experimental import pallas as pl


def kernel(x, edge_index, edge_type, comp1, bases1, root1, bias1, comp2, bases2, root2, bias2):
    raise NotImplementedError("write your pallas kernel here")


# speedup vs baseline: 1.0343x; 1.0343x over previous
import jax
import jax.numpy as jnp
from jax.experimental import pallas as pl


def _combine3_relu(a_ref, b_ref, c_ref, o_ref):
    o_ref[...] = jnp.maximum(a_ref[...] + b_ref[...] + c_ref[...], 0.0)


def _combine2(a_ref, b_ref, o_ref):
    o_ref[...] = a_ref[...] + b_ref[...]


def kernel(x, edge_index, edge_type, comp1, bases1, root1, bias1, comp2, bases2, root2, bias2):
    n = x.shape[0]
    R = comp1.shape[0]
    src, dst = edge_index[0], edge_index[1]
    key = edge_type * n + dst
    cnt = jnp.zeros((R * n,), x.dtype).at[key].add(1.0)
    recip = (1.0 / jnp.maximum(cnt, 1.0))[key]
    gkey = edge_type * n + src

    def conv(h, comp, bases, root):
        W = jnp.einsum('rb,bio->rio', comp, bases)
        y = jnp.einsum('ni,rio->rno', h, W).reshape(R * n, -1)
        msgs = y[gkey] * recip[:, None]
        agg = jnp.zeros((n, y.shape[-1]), h.dtype).at[dst].add(msgs)
        return agg, h @ root

    a1, z1 = conv(x, comp1, bases1, root1)
    b1 = jnp.broadcast_to(bias1[None, :], a1.shape)
    h = pl.pallas_call(
        _combine3_relu,
        out_shape=jax.ShapeDtypeStruct(a1.shape, a1.dtype),
    )(a1, z1 + b1, jnp.zeros_like(a1))
    a2, z2 = conv(h, comp2, bases2, root2)
    out = pl.pallas_call(
        _combine2,
        out_shape=jax.ShapeDtypeStruct(a2.shape, a2.dtype),
    )(a2, z2 + bias2[None, :])
    return out


# speedup vs baseline: 9.1193x; 8.8169x over previous
import functools

import jax
import jax.numpy as jnp
from jax import lax
from jax.experimental import pallas as pl
from jax.experimental.pallas import tpu as pltpu
from jax.experimental.pallas import tpu_sc as plsc

N = 10000
E = 320000
R = 8
NB = 4
D = 128

NC = 2
NS = 16
L = 16

C = 80
ROWS = E // C
EPT = E // NC // NS
CPT = EPT // C
CNT_PASSES = 2
RN = R * N
BUF = 64
SPANS = ((0, BUF), (BUF, CPT - BUF))

_mesh = plsc.VectorSubcoreMesh(
    core_axis_name="c", subcore_axis_name="s", num_cores=NC, num_subcores=NS)
_sc_params = pltpu.CompilerParams(needs_layout_passes=False)


_ZB = RN // NS + 8


def _count_body(src_h, dst_h, et_h, recip_h, gkey_h,
                cnt_sh, cnt_loc, a_v, b_v, rec_v, ones_v, zb_v):
  c = lax.axis_index("c")
  s = lax.axis_index("s")

  z16 = jnp.zeros((L,), jnp.float32)

  def zrow(i, _):
    zb_v[pl.ds(i * L, L)] = z16
    return 0
  lax.fori_loop(0, _ZB // L, zrow, 0)
  zsl = pl.ds(s * (RN // NS), RN // NS)
  pltpu.sync_copy(zb_v.at[pl.ds(0, RN // NS)], cnt_sh.at[zsl])
  for j in range(C // L):
    ones_v[pl.ds(j * L, L)] = jnp.full((L,), 1.0, jnp.float32)
  plsc.subcore_barrier()

  for p in range(CNT_PASSES):
    blk = CNT_PASSES * s + p
    for off, nr in SPANS:
      pltpu.sync_copy(et_h.at[blk, pl.ds(off, nr)], a_v.at[pl.ds(0, nr)])
      pltpu.sync_copy(dst_h.at[blk, pl.ds(off, nr)], b_v.at[pl.ds(0, nr)])

      def key_row(i, _):
        for j in range(C // L):
          sl = (i, pl.ds(j * L, L))
          a_v[sl] = a_v[sl] * N + b_v[sl]
        return 0
      lax.fori_loop(0, nr, key_row, 0)

      def scat_row(i, _):
        pltpu.sync_copy(ones_v, cnt_sh.at[a_v.at[i]], add=True)
        return 0
      lax.fori_loop(0, nr, scat_row, 0)

  plsc.subcore_barrier()
  pltpu.sync_copy(cnt_sh, cnt_loc)

  wid = c * NS + s
  for off, nr in SPANS:
    pltpu.sync_copy(et_h.at[wid, pl.ds(off, nr)], a_v.at[pl.ds(0, nr)])
    pltpu.sync_copy(dst_h.at[wid, pl.ds(off, nr)], b_v.at[pl.ds(0, nr)])

    def recip_row(i, _):
      for j in range(C // L):
        sl = (i, pl.ds(j * L, L))
        k16 = a_v[sl] * N + b_v[sl]
        g = plsc.load_gather(cnt_loc, [k16])
        rec_v[sl] = 1.0 / jnp.maximum(g, 1.0)
      return 0
    lax.fori_loop(0, nr, recip_row, 0)
    pltpu.sync_copy(rec_v.at[pl.ds(0, nr)], recip_h.at[wid, pl.ds(off, nr)])

    pltpu.sync_copy(src_h.at[wid, pl.ds(off, nr)], b_v.at[pl.ds(0, nr)])

    def gkey_row(i, _):
      for j in range(C // L):
        sl = (i, pl.ds(j * L, L))
        a_v[sl] = a_v[sl] * N + b_v[sl]
      return 0
    lax.fori_loop(0, nr, gkey_row, 0)
    pltpu.sync_copy(a_v.at[pl.ds(0, nr)], gkey_h.at[wid, pl.ds(off, nr)])


_count_kernel = pl.kernel(
    _count_body,
    out_type=[jax.ShapeDtypeStruct((NC * NS, CPT, C), jnp.float32),
              jax.ShapeDtypeStruct((NC * NS, CPT, C), jnp.int32)],
    mesh=_mesh,
    scratch_types=[
        pltpu.MemorySpace.VMEM_SHARED((RN,), jnp.float32),
        pltpu.VMEM((RN,), jnp.float32),
        pltpu.VMEM((BUF, C), jnp.int32),
        pltpu.VMEM((BUF, C), jnp.int32),
        pltpu.VMEM((BUF, C), jnp.float32),
        pltpu.VMEM((C,), jnp.float32),
        pltpu.VMEM((_ZB,), jnp.float32),
    ],
    compiler_params=_sc_params,
)


def _scatter_body(y_h, gkey_h, dst_h, rec_h, acc_h,
                  acc_sh, key_v, dst_v, rec_v, rows_v, sem):
  c = lax.axis_index("c")
  s = lax.axis_index("s")

  z16 = jnp.zeros((L,), jnp.float32)

  def zrow(i, _):
    for j in range(D // L):
      rows_v[i, pl.ds(j * L, L)] = z16
    return 0
  lax.fori_loop(0, C, zrow, 0)
  z0 = pl.multiple_of((s * (N // NS)) // 8 * 8, 8)
  for k in range(8):
    sz = C if k < 7 else 632 - 7 * C
    pltpu.sync_copy(rows_v.at[pl.ds(0, sz)],
                    acc_sh.at[pl.ds(z0 + k * C, sz)])

  wid = c * NS + s
  plsc.subcore_barrier()

  for off, nr in SPANS:
    pltpu.sync_copy(gkey_h.at[wid, pl.ds(off, nr)], key_v.at[pl.ds(0, nr)])
    pltpu.sync_copy(dst_h.at[wid, pl.ds(off, nr)], dst_v.at[pl.ds(0, nr)])
    pltpu.sync_copy(rec_h.at[wid, pl.ds(off, nr)], rec_v.at[pl.ds(0, nr)])

    def chunk(i, _):
      pltpu.async_copy(y_h.at[key_v.at[i]], rows_v, sem).wait()

      def scale(g, _):
        rec16 = rec_v[i, pl.ds(g * L, L)]
        for e16 in range(L):
          rv = lax.broadcast(rec16[e16], (L,))
          e = g * L + e16
          for j in range(D // L):
            sl = (e, pl.ds(j * L, L))
            rows_v[sl] = rows_v[sl] * rv
        return 0
      lax.fori_loop(0, C // L, scale, 0)

      pltpu.sync_copy(rows_v, acc_sh.at[dst_v.at[i]], add=True)
      return 0
    lax.fori_loop(0, nr, chunk, 0)

  plsc.subcore_barrier()
  for k in range(8):
    sz = C if k < 7 else 632 - 7 * C
    sl = pl.ds(z0 + k * C, sz)
    pltpu.sync_copy(acc_sh.at[sl], rows_v.at[pl.ds(0, sz)])
    pltpu.sync_copy(rows_v.at[pl.ds(0, sz)], acc_h.at[c, sl])


_scatter_kernel = pl.kernel(
    _scatter_body,
    out_type=jax.ShapeDtypeStruct((NC, N, D), jnp.float32),
    mesh=_mesh,
    scratch_types=[
        pltpu.MemorySpace.VMEM_SHARED((N, D), jnp.float32),
        pltpu.VMEM((BUF, C), jnp.int32),
        pltpu.VMEM((BUF, C), jnp.int32),
        pltpu.VMEM((BUF, C), jnp.float32),
        pltpu.VMEM((C, D), jnp.float32),
        pltpu.SemaphoreType.DMA,
    ],
    compiler_params=_sc_params,
)


def _edges3d(a):
  return a.reshape(NC * NS, CPT, C)


BN = 1000
NBLK = N // BN


def _mm1_body(comp_s, bases_v, root_v, bias_v, x_v, y_ref):
  r = pl.program_id(1)

  @pl.when(r < R)
  def _():
    w = (comp_s[r, 0] * bases_v[0] + comp_s[r, 1] * bases_v[1]
         + comp_s[r, 2] * bases_v[2] + comp_s[r, 3] * bases_v[3])
    y_ref[0] = jnp.dot(x_v[...], w, preferred_element_type=jnp.float32)

  @pl.when(r == R)
  def _():
    y_ref[0] = (jnp.dot(x_v[...], root_v[...],
                        preferred_element_type=jnp.float32) + bias_v[...])


def _mm1(x, comp, bases, root, bias):
  return pl.pallas_call(
      _mm1_body,
      grid=(NBLK, R + 1),
      in_specs=[
          pl.BlockSpec(memory_space=pltpu.MemorySpace.SMEM),
          pl.BlockSpec((NB, D, D), lambda nb, r: (0, 0, 0)),
          pl.BlockSpec((D, D), lambda nb, r: (0, 0)),
          pl.BlockSpec((1, D), lambda nb, r: (0, 0)),
          pl.BlockSpec((BN, D), lambda nb, r: (nb, 0)),
      ],
      out_specs=pl.BlockSpec((1, BN, D), lambda nb, r: (r, nb, 0)),
      out_shape=jax.ShapeDtypeStruct((R + 1, N, D), jnp.float32),
  )(comp, bases, root, bias.reshape(1, D), x)


def _mm2_body(comp_s, bases_v, root_v, bias_v, acc_v, z_v, y_ref, h_v):
  r = pl.program_id(1)

  @pl.when(r == 0)
  def _():
    h_v[...] = jnp.maximum(acc_v[0] + acc_v[1] + z_v[0], 0.0)

  @pl.when(r < R)
  def _():
    w = (comp_s[r, 0] * bases_v[0] + comp_s[r, 1] * bases_v[1]
         + comp_s[r, 2] * bases_v[2] + comp_s[r, 3] * bases_v[3])
    y_ref[0] = jnp.dot(h_v[...], w, preferred_element_type=jnp.float32)

  @pl.when(r == R)
  def _():
    y_ref[0] = (jnp.dot(h_v[...], root_v[...],
                        preferred_element_type=jnp.float32) + bias_v[...])


def _mm2(acc, z_stack, comp, bases, root, bias):
  return pl.pallas_call(
      _mm2_body,
      grid=(NBLK, R + 1),
      in_specs=[
          pl.BlockSpec(memory_space=pltpu.MemorySpace.SMEM),
          pl.BlockSpec((NB, D, D), lambda nb, r: (0, 0, 0)),
          pl.BlockSpec((D, D), lambda nb, r: (0, 0)),
          pl.BlockSpec((1, D), lambda nb, r: (0, 0)),
          pl.BlockSpec((NC, BN, D), lambda nb, r: (0, nb, 0)),
          pl.BlockSpec((1, BN, D), lambda nb, r: (R, nb, 0)),
      ],
      out_specs=pl.BlockSpec((1, BN, D), lambda nb, r: (r, nb, 0)),
      out_shape=jax.ShapeDtypeStruct((R + 1, N, D), jnp.float32),
      scratch_shapes=[pltpu.VMEM((BN, D), jnp.float32)],
  )(comp, bases, root, bias.reshape(1, D), acc, z_stack)


def _final_body(acc_v, z_v, o_ref):
  o_ref[...] = acc_v[0] + acc_v[1] + z_v[0]


def _final(acc, z_stack):
  return pl.pallas_call(
      _final_body,
      grid=(NBLK,),
      in_specs=[
          pl.BlockSpec((NC, BN, D), lambda nb: (0, nb, 0)),
          pl.BlockSpec((1, BN, D), lambda nb: (R, nb, 0)),
      ],
      out_specs=pl.BlockSpec((BN, D), lambda nb: (nb, 0)),
      out_shape=jax.ShapeDtypeStruct((N, D), jnp.float32),
  )(acc, z_stack)


def kernel(x, edge_index, edge_type, comp1, bases1, root1, bias1,
           comp2, bases2, root2, bias2):
  src2d = _edges3d(edge_index[0])
  dst2d = _edges3d(edge_index[1])
  et2d = _edges3d(edge_type)

  recip2d, gkey2d = _count_kernel(src2d, dst2d, et2d)

  yz1 = _mm1(x, comp1, bases1, root1, bias1)
  acc1 = _scatter_kernel(yz1.reshape((R + 1) * N, D), gkey2d, dst2d, recip2d)
  yz2 = _mm2(acc1, yz1, comp2, bases2, root2, bias2)
  acc2 = _scatter_kernel(yz2.reshape((R + 1) * N, D), gkey2d, dst2d, recip2d)
  return _final(acc2, yz2)


# speedup vs baseline: 13.1574x; 1.4428x over previous
import functools

import jax
import jax.numpy as jnp
from jax import lax
from jax.experimental import pallas as pl
from jax.experimental.pallas import tpu as pltpu
from jax.experimental.pallas import tpu_sc as plsc

N = 10000
E = 320000
R = 8
NB = 4
D = 128

NC = 2
NS = 16
L = 16

C = 80
ROWS = E // C
EPT = E // NC // NS
CPT = EPT // C
CNT_PASSES = 2
RN = R * N
BUF = 64
SPANS = ((0, BUF), (BUF, CPT - BUF))

_mesh = plsc.VectorSubcoreMesh(
    core_axis_name="c", subcore_axis_name="s", num_cores=NC, num_subcores=NS)
_sc_params = pltpu.CompilerParams(needs_layout_passes=False)


_ZB = RN // NS + 8


def _count_body(src_h, dst_h, et_h, recip_h, gkey_h,
                cnt_sh, cnt_loc, a_v, b_v, rec_v, ones_v, zb_v):
  c = lax.axis_index("c")
  s = lax.axis_index("s")

  z16 = jnp.zeros((L,), jnp.float32)

  def zrow(i, _):
    zb_v[pl.ds(i * L, L)] = z16
    return 0
  lax.fori_loop(0, _ZB // L, zrow, 0)
  zsl = pl.ds(s * (RN // NS), RN // NS)
  pltpu.sync_copy(zb_v.at[pl.ds(0, RN // NS)], cnt_sh.at[zsl])
  for j in range(C // L):
    ones_v[pl.ds(j * L, L)] = jnp.full((L,), 1.0, jnp.float32)
  plsc.subcore_barrier()

  for p in range(CNT_PASSES):
    blk = CNT_PASSES * s + p
    for off, nr in SPANS:
      pltpu.sync_copy(et_h.at[blk, pl.ds(off, nr)], a_v.at[pl.ds(0, nr)])
      pltpu.sync_copy(dst_h.at[blk, pl.ds(off, nr)], b_v.at[pl.ds(0, nr)])

      def key_row(i, _):
        for j in range(C // L):
          sl = (i, pl.ds(j * L, L))
          a_v[sl] = a_v[sl] * N + b_v[sl]
        return 0
      lax.fori_loop(0, nr, key_row, 0)

      def scat_row(i, _):
        pltpu.sync_copy(ones_v, cnt_sh.at[a_v.at[i]], add=True)
        return 0
      lax.fori_loop(0, nr, scat_row, 0)

  plsc.subcore_barrier()
  pltpu.sync_copy(cnt_sh, cnt_loc)

  wid = c * NS + s
  for off, nr in SPANS:
    pltpu.sync_copy(et_h.at[wid, pl.ds(off, nr)], a_v.at[pl.ds(0, nr)])
    pltpu.sync_copy(dst_h.at[wid, pl.ds(off, nr)], b_v.at[pl.ds(0, nr)])

    def recip_row(i, _):
      for j in range(C // L):
        sl = (i, pl.ds(j * L, L))
        k16 = a_v[sl] * N + b_v[sl]
        g = plsc.load_gather(cnt_loc, [k16])
        rec_v[sl] = 1.0 / jnp.maximum(g, 1.0)
      return 0
    lax.fori_loop(0, nr, recip_row, 0)
    pltpu.sync_copy(rec_v.at[pl.ds(0, nr)], recip_h.at[wid, pl.ds(off, nr)])

    pltpu.sync_copy(src_h.at[wid, pl.ds(off, nr)], b_v.at[pl.ds(0, nr)])

    def gkey_row(i, _):
      for j in range(C // L):
        sl = (i, pl.ds(j * L, L))
        a_v[sl] = a_v[sl] * N + b_v[sl]
      return 0
    lax.fori_loop(0, nr, gkey_row, 0)
    pltpu.sync_copy(a_v.at[pl.ds(0, nr)], gkey_h.at[wid, pl.ds(off, nr)])


_count_kernel = pl.kernel(
    _count_body,
    out_type=[jax.ShapeDtypeStruct((NC * NS, CPT, C), jnp.float32),
              jax.ShapeDtypeStruct((NC * NS, CPT, C), jnp.int32)],
    mesh=_mesh,
    scratch_types=[
        pltpu.MemorySpace.VMEM_SHARED((RN,), jnp.float32),
        pltpu.VMEM((RN,), jnp.float32),
        pltpu.VMEM((BUF, C), jnp.int32),
        pltpu.VMEM((BUF, C), jnp.int32),
        pltpu.VMEM((BUF, C), jnp.float32),
        pltpu.VMEM((C,), jnp.float32),
        pltpu.VMEM((_ZB,), jnp.float32),
    ],
    compiler_params=_sc_params,
)


def _scatter_body(y_h, gkey_h, dst_h, rec_h, acc_h,
                  acc_sh, key_v, dst_v, rec_v, rows0_v, rows1_v,
                  gsem0, gsem1, ssem0, ssem1):
  c = lax.axis_index("c")
  s = lax.axis_index("s")

  z16 = jnp.zeros((L,), jnp.float32)

  def zrow(i, _):
    for j in range(D // L):
      rows0_v[i, pl.ds(j * L, L)] = z16
    return 0
  lax.fori_loop(0, C, zrow, 0)
  z0 = pl.multiple_of((s * (N // NS)) // 8 * 8, 8)
  for k in range(8):
    sz = C if k < 7 else 632 - 7 * C
    pltpu.sync_copy(rows0_v.at[pl.ds(0, sz)],
                    acc_sh.at[pl.ds(z0 + k * C, sz)])

  wid = c * NS + s
  plsc.subcore_barrier()

  def start_gather(r, buf, sem):
    pltpu.async_copy(y_h.at[key_v.at[r]], buf, sem)

  def wait_dma(buf, sem):
    pltpu.make_async_copy(y_h.at[pl.ds(0, C)], buf, sem).wait()

  def scale(r, buf):
    def grp(g, _):
      rec16 = rec_v[r, pl.ds(g * L, L)]
      for e16 in range(L):
        rv = lax.broadcast(rec16[e16], (L,))
        for j in range(D // L):
          sl = (g * L + e16, pl.ds(j * L, L))
          buf[sl] = buf[sl] * rv
      return 0
    lax.fori_loop(0, C // L, grp, 0)

  def start_scat(r, buf, sem):
    pltpu.async_copy(buf, acc_sh.at[dst_v.at[r]], sem, add=True)

  for off, nr in SPANS:
    pltpu.sync_copy(gkey_h.at[wid, pl.ds(off, nr)], key_v.at[pl.ds(0, nr)])
    pltpu.sync_copy(dst_h.at[wid, pl.ds(off, nr)], dst_v.at[pl.ds(0, nr)])
    pltpu.sync_copy(rec_h.at[wid, pl.ds(off, nr)], rec_v.at[pl.ds(0, nr)])

    start_gather(0, rows0_v, gsem0)

    def pair(i, _):
      @pl.when(i > 0)
      def _():
        wait_dma(rows1_v, ssem1)
      start_gather(2 * i + 1, rows1_v, gsem1)
      wait_dma(rows0_v, gsem0)
      scale(2 * i, rows0_v)
      start_scat(2 * i, rows0_v, ssem0)
      wait_dma(rows1_v, gsem1)
      scale(2 * i + 1, rows1_v)
      wait_dma(rows0_v, ssem0)

      @pl.when(2 * i + 2 < nr)
      def _():
        start_gather(2 * i + 2, rows0_v, gsem0)
      start_scat(2 * i + 1, rows1_v, ssem1)
      return 0
    lax.fori_loop(0, nr // 2, pair, 0)
    wait_dma(rows1_v, ssem1)

    if nr % 2:
      wait_dma(rows0_v, gsem0)
      scale(nr - 1, rows0_v)
      pltpu.sync_copy(rows0_v, acc_sh.at[dst_v.at[nr - 1]], add=True)

  plsc.subcore_barrier()
  for k in range(8):
    sz = C if k < 7 else 632 - 7 * C
    sl = pl.ds(z0 + k * C, sz)
    pltpu.sync_copy(acc_sh.at[sl], rows0_v.at[pl.ds(0, sz)])
    pltpu.sync_copy(rows0_v.at[pl.ds(0, sz)], acc_h.at[c, sl])


_scatter_kernel = pl.kernel(
    _scatter_body,
    out_type=jax.ShapeDtypeStruct((NC, N, D), jnp.float32),
    mesh=_mesh,
    scratch_types=[
        pltpu.MemorySpace.VMEM_SHARED((N, D), jnp.float32),
        pltpu.VMEM((BUF, C), jnp.int32),
        pltpu.VMEM((BUF, C), jnp.int32),
        pltpu.VMEM((BUF, C), jnp.float32),
        pltpu.VMEM((C, D), jnp.float32),
        pltpu.VMEM((C, D), jnp.float32),
        pltpu.SemaphoreType.DMA,
        pltpu.SemaphoreType.DMA,
        pltpu.SemaphoreType.DMA,
        pltpu.SemaphoreType.DMA,
    ],
    compiler_params=_sc_params,
)


def _edges3d(a):
  return a.reshape(NC * NS, CPT, C)


BN = 1000
NBLK = N // BN


def _mm1_body(comp_s, bases_v, root_v, bias_v, x_v, y_ref):
  r = pl.program_id(1)

  @pl.when(r < R)
  def _():
    w = (comp_s[r, 0] * bases_v[0] + comp_s[r, 1] * bases_v[1]
         + comp_s[r, 2] * bases_v[2] + comp_s[r, 3] * bases_v[3])
    y_ref[0] = jnp.dot(x_v[...], w, preferred_element_type=jnp.float32)

  @pl.when(r == R)
  def _():
    y_ref[0] = (jnp.dot(x_v[...], root_v[...],
                        preferred_element_type=jnp.float32) + bias_v[...])


def _mm1(x, comp, bases, root, bias):
  return pl.pallas_call(
      _mm1_body,
      grid=(NBLK, R + 1),
      in_specs=[
          pl.BlockSpec(memory_space=pltpu.MemorySpace.SMEM),
          pl.BlockSpec((NB, D, D), lambda nb, r: (0, 0, 0)),
          pl.BlockSpec((D, D), lambda nb, r: (0, 0)),
          pl.BlockSpec((1, D), lambda nb, r: (0, 0)),
          pl.BlockSpec((BN, D), lambda nb, r: (nb, 0)),
      ],
      out_specs=pl.BlockSpec((1, BN, D), lambda nb, r: (r, nb, 0)),
      out_shape=jax.ShapeDtypeStruct((R + 1, N, D), jnp.float32),
  )(comp, bases, root, bias.reshape(1, D), x)


def _mm2_body(comp_s, bases_v, root_v, bias_v, acc_v, z_v, y_ref, h_v):
  r = pl.program_id(1)

  @pl.when(r == 0)
  def _():
    h_v[...] = jnp.maximum(acc_v[0] + acc_v[1] + z_v[0], 0.0)

  @pl.when(r < R)
  def _():
    w = (comp_s[r, 0] * bases_v[0] + comp_s[r, 1] * bases_v[1]
         + comp_s[r, 2] * bases_v[2] + comp_s[r, 3] * bases_v[3])
    y_ref[0] = jnp.dot(h_v[...], w, preferred_element_type=jnp.float32)

  @pl.when(r == R)
  def _():
    y_ref[0] = (jnp.dot(h_v[...], root_v[...],
                        preferred_element_type=jnp.float32) + bias_v[...])


def _mm2(acc, z_stack, comp, bases, root, bias):
  return pl.pallas_call(
      _mm2_body,
      grid=(NBLK, R + 1),
      in_specs=[
          pl.BlockSpec(memory_space=pltpu.MemorySpace.SMEM),
          pl.BlockSpec((NB, D, D), lambda nb, r: (0, 0, 0)),
          pl.BlockSpec((D, D), lambda nb, r: (0, 0)),
          pl.BlockSpec((1, D), lambda nb, r: (0, 0)),
          pl.BlockSpec((NC, BN, D), lambda nb, r: (0, nb, 0)),
          pl.BlockSpec((1, BN, D), lambda nb, r: (R, nb, 0)),
      ],
      out_specs=pl.BlockSpec((1, BN, D), lambda nb, r: (r, nb, 0)),
      out_shape=jax.ShapeDtypeStruct((R + 1, N, D), jnp.float32),
      scratch_shapes=[pltpu.VMEM((BN, D), jnp.float32)],
  )(comp, bases, root, bias.reshape(1, D), acc, z_stack)


def _final_body(acc_v, z_v, o_ref):
  o_ref[...] = acc_v[0] + acc_v[1] + z_v[0]


def _final(acc, z_stack):
  return pl.pallas_call(
      _final_body,
      grid=(NBLK,),
      in_specs=[
          pl.BlockSpec((NC, BN, D), lambda nb: (0, nb, 0)),
          pl.BlockSpec((1, BN, D), lambda nb: (R, nb, 0)),
      ],
      out_specs=pl.BlockSpec((BN, D), lambda nb: (nb, 0)),
      out_shape=jax.ShapeDtypeStruct((N, D), jnp.float32),
  )(acc, z_stack)


def kernel(x, edge_index, edge_type, comp1, bases1, root1, bias1,
           comp2, bases2, root2, bias2):
  src2d = _edges3d(edge_index[0])
  dst2d = _edges3d(edge_index[1])
  et2d = _edges3d(edge_type)

  recip2d, gkey2d = _count_kernel(src2d, dst2d, et2d)

  yz1 = _mm1(x, comp1, bases1, root1, bias1)
  acc1 = _scatter_kernel(yz1.reshape((R + 1) * N, D), gkey2d, dst2d, recip2d)
  yz2 = _mm2(acc1, yz1, comp2, bases2, root2, bias2)
  acc2 = _scatter_kernel(yz2.reshape((R + 1) * N, D), gkey2d, dst2d, recip2d)
  return _final(acc2, yz2)
